# Initial kernel scaffold; baseline (speedup 1.0000x reference)
import jax
import jax.numpy as jnp
from jax.experimental import pallas as pl


def kernel(representation, atom_mask, W_ih, W_hh, b_ih, b_hh, W_out, b_out, mean, stddev):
    raise NotImplementedError("write your pallas kernel here")


# speedup vs baseline: 1.3005x; 1.3005x over previous
import functools

import jax
import jax.numpy as jnp
from jax.experimental import pallas as pl

_B, _N, _D = 128, 1024, 128
_STEPS = 3
_BB = 8


def _body(x_ref, maskf_ref, w_ih_t_ref, w_hh_t_ref, b2_ref, w_out_t_ref,
          b_out_ref, scale_ref, y_ref):
    i = pl.program_id(0)
    x = x_ref[...]
    maskf = maskf_ref[...]
    w_ih_t = w_ih_t_ref[...]
    w_hh_t = w_hh_t_ref[...]
    b2 = b2_ref[...]

    d = _D
    h = jnp.zeros((_BB, d), dtype=jnp.float32)
    c = jnp.zeros((_BB, d), dtype=jnp.float32)
    q_star = jnp.zeros((_BB, 2 * d), dtype=jnp.float32)
    neg_big = jnp.float32(-jnp.inf)

    for _ in range(_STEPS):
        gates = (
            jax.lax.dot_general(q_star, w_ih_t, (((1,), (0,)), ((), ())),
                                preferred_element_type=jnp.float32)
            + jax.lax.dot_general(h, w_hh_t, (((1,), (0,)), ((), ())),
                                  preferred_element_type=jnp.float32)
            + b2
        )
        ig = jax.nn.sigmoid(gates[:, 0 * d:1 * d])
        fg = jax.nn.sigmoid(gates[:, 1 * d:2 * d])
        gg = jnp.tanh(gates[:, 2 * d:3 * d])
        og = jax.nn.sigmoid(gates[:, 3 * d:4 * d])
        c = fg * c + ig * gg
        h = og * jnp.tanh(c)
        q = h

        e = jax.lax.dot_general(x, q, (((2,), (1,)), ((0,), (0,))),
                                preferred_element_type=jnp.float32)
        e = jnp.where(maskf > 0, e, neg_big)
        m = jnp.max(e, axis=1, keepdims=True)
        a = jnp.exp(e - m) * maskf
        s = jnp.sum(a, axis=1, keepdims=True)
        r = jax.lax.dot_general(a, x, (((1,), (1,)), ((0,), (0,))),
                                preferred_element_type=jnp.float32)
        r = r / s
        q_star = jnp.concatenate([q, r], axis=1)

    y = jax.lax.dot_general(q_star, w_out_t_ref[...], (((1,), (0,)), ((), ())),
                            preferred_element_type=jnp.float32)
    y = y + b_out_ref[...]
    y = y * scale_ref[0, 1] + scale_ref[0, 0]
    y_ref[pl.ds(i * _BB, _BB), :] = y


@jax.jit
def kernel(representation, atom_mask, W_ih, W_hh, b_ih, b_hh, W_out, b_out,
           mean, stddev):
    maskf = atom_mask.astype(jnp.float32)
    w_ih_t = W_ih.T
    w_hh_t = W_hh.T
    b2 = (b_ih + b_hh).reshape(1, 4 * _D)
    w_out_t = W_out.T
    b_out2 = b_out.reshape(1, 1)
    scale = jnp.stack([mean[0], stddev[0]]).reshape(1, 2)

    grid = (_B // _BB,)
    y = pl.pallas_call(
        _body,
        grid=grid,
        in_specs=[
            pl.BlockSpec((_BB, _N, _D), lambda i: (i, 0, 0)),
            pl.BlockSpec((_BB, _N), lambda i: (i, 0)),
            pl.BlockSpec((2 * _D, 4 * _D), lambda i: (0, 0)),
            pl.BlockSpec((_D, 4 * _D), lambda i: (0, 0)),
            pl.BlockSpec((1, 4 * _D), lambda i: (0, 0)),
            pl.BlockSpec((2 * _D, 1), lambda i: (0, 0)),
            pl.BlockSpec((1, 1), lambda i: (0, 0)),
            pl.BlockSpec((1, 2), lambda i: (0, 0)),
        ],
        out_specs=pl.BlockSpec((_B, 1), lambda i: (0, 0)),
        out_shape=jax.ShapeDtypeStruct((_B, 1), jnp.float32),
    )(representation, maskf, w_ih_t, w_hh_t, b2, w_out_t, b_out2, scale)
    return y


# speedup vs baseline: 1.4042x; 1.0798x over previous
import functools

import jax
import jax.numpy as jnp
from jax.experimental import pallas as pl

_B, _N, _D = 128, 1024, 128
_STEPS = 3
_BB = 16


def _body(x_ref, maskf_ref, w_ih_t_ref, w_hh_t_ref, b2_ref, w_out_t_ref,
          b_out_ref, scale_ref, y_ref):
    i = pl.program_id(0)
    x = x_ref[...]
    maskf = maskf_ref[...]
    w_ih_t = w_ih_t_ref[...]
    w_hh_t = w_hh_t_ref[...]
    b2 = b2_ref[...]

    d = _D
    h = jnp.zeros((_BB, d), dtype=jnp.float32)
    c = jnp.zeros((_BB, d), dtype=jnp.float32)
    q_star = jnp.zeros((_BB, 2 * d), dtype=jnp.float32)
    neg_big = jnp.float32(-jnp.inf)

    for _ in range(_STEPS):
        gates = (
            jax.lax.dot_general(q_star, w_ih_t, (((1,), (0,)), ((), ())),
                                preferred_element_type=jnp.float32)
            + jax.lax.dot_general(h, w_hh_t, (((1,), (0,)), ((), ())),
                                  preferred_element_type=jnp.float32)
            + b2
        )
        ig = jax.nn.sigmoid(gates[:, 0 * d:1 * d])
        fg = jax.nn.sigmoid(gates[:, 1 * d:2 * d])
        gg = jnp.tanh(gates[:, 2 * d:3 * d])
        og = jax.nn.sigmoid(gates[:, 3 * d:4 * d])
        c = fg * c + ig * gg
        h = og * jnp.tanh(c)
        q = h

        e = jax.lax.dot_general(x, q, (((2,), (1,)), ((0,), (0,))),
                                preferred_element_type=jnp.float32)
        e = jnp.where(maskf > 0, e, neg_big)
        m = jnp.max(e, axis=1, keepdims=True)
        a = jnp.exp(e - m) * maskf
        s = jnp.sum(a, axis=1, keepdims=True)
        r = jax.lax.dot_general(a, x, (((1,), (1,)), ((0,), (0,))),
                                preferred_element_type=jnp.float32)
        r = r / s
        q_star = jnp.concatenate([q, r], axis=1)

    y = jax.lax.dot_general(q_star, w_out_t_ref[...], (((1,), (0,)), ((), ())),
                            preferred_element_type=jnp.float32)
    y = y + b_out_ref[...]
    y = y * scale_ref[0, 1] + scale_ref[0, 0]
    y_ref[pl.ds(i * _BB, _BB), :] = y


@jax.jit
def kernel(representation, atom_mask, W_ih, W_hh, b_ih, b_hh, W_out, b_out,
           mean, stddev):
    maskf = atom_mask.astype(jnp.float32)
    w_ih_t = W_ih.T
    w_hh_t = W_hh.T
    b2 = (b_ih + b_hh).reshape(1, 4 * _D)
    w_out_t = W_out.T
    b_out2 = b_out.reshape(1, 1)
    scale = jnp.stack([mean[0], stddev[0]]).reshape(1, 2)

    grid = (_B // _BB,)
    y = pl.pallas_call(
        _body,
        grid=grid,
        in_specs=[
            pl.BlockSpec((_BB, _N, _D), lambda i: (i, 0, 0)),
            pl.BlockSpec((_BB, _N), lambda i: (i, 0)),
            pl.BlockSpec((2 * _D, 4 * _D), lambda i: (0, 0)),
            pl.BlockSpec((_D, 4 * _D), lambda i: (0, 0)),
            pl.BlockSpec((1, 4 * _D), lambda i: (0, 0)),
            pl.BlockSpec((2 * _D, 1), lambda i: (0, 0)),
            pl.BlockSpec((1, 1), lambda i: (0, 0)),
            pl.BlockSpec((1, 2), lambda i: (0, 0)),
        ],
        out_specs=pl.BlockSpec((_B, 1), lambda i: (0, 0)),
        out_shape=jax.ShapeDtypeStruct((_B, 1), jnp.float32),
    )(representation, maskf, w_ih_t, w_hh_t, b2, w_out_t, b_out2, scale)
    return y
